# baseline (speedup 1.0000x reference)
import functools

import jax
import jax.numpy as jnp
from jax import lax
from jax.experimental import pallas as pl
from jax.experimental.pallas import tpu as pltpu
from jax.experimental.pallas import tpu_sc as plsc

_NW = 32
_ST = 25
_BT = 128
_BTW = 4
_STC = 5


@jax.jit
def _embed(xq, wcols):
    d = wcols.shape[0]
    mesh = plsc.VectorSubcoreMesh(core_axis_name="c", subcore_axis_name="s")

    n_stc = _ST // _STC
    n_chunks = _BTW * n_stc

    @functools.partial(
        pl.kernel,
        out_type=jax.ShapeDtypeStruct((d, _ST, _BT, 8, 128), jnp.float32),
        mesh=mesh,
        scratch_types=[
            pltpu.VMEM((2, _STC, 8, 128), jnp.int32),
            pltpu.VMEM((2, d, _STC, 8, 128), jnp.float32),
            pltpu.VMEM((d, 16), jnp.float32),
            pltpu.SemaphoreType.DMA,
            pltpu.SemaphoreType.DMA,
            pltpu.SemaphoreType.DMA,
            pltpu.SemaphoreType.DMA,
        ],
        compiler_params=pltpu.CompilerParams(
            use_tc_tiling_on_sc=False, needs_layout_passes=False
        ),
    )
    def k(w_hbm, x_hbm, out_hbm, xv, outv, wv, si0, si1, so0, so1):
        in_sems = (si0, si1)
        out_sems = (so0, so1)
        wid = lax.axis_index("s") * 2 + lax.axis_index("c")
        bt0 = wid * _BTW
        pltpu.sync_copy(w_hbm, wv)
        wc = [wv[c] for c in range(d)]

        def x_src(ci):
            bt = bt0 + ci // n_stc
            st0 = (ci % n_stc) * _STC
            return x_hbm.at[pl.ds(st0, _STC), bt]

        def out_dst(ci):
            bt = bt0 + ci // n_stc
            st0 = (ci % n_stc) * _STC
            return out_hbm.at[:, pl.ds(st0, _STC), bt]

        for b in range(2):
            pltpu.make_async_copy(x_src(b), xv.at[b], in_sems[b]).start()

        def body(j, carry):
            for b in range(2):
                ci = 2 * j + b

                @pl.when(ci >= 2)
                def _():
                    pltpu.make_async_copy(
                        outv.at[b], out_dst(ci - 2), out_sems[b]
                    ).wait()

                pltpu.make_async_copy(x_src(ci), xv.at[b], in_sems[b]).wait()

                @plsc.parallel_loop(0, _STC * 8, unroll=4)
                def st_body(g):
                    sl = g // 8
                    si = g % 8
                    for bq in range(8):
                        xs = xv[b, sl, si, pl.ds(bq * 16, 16)]
                        for c in range(d):
                            vals = jnp.take_along_axis(wc[c], xs, axis=0)
                            outv[b, c, sl, si, pl.ds(bq * 16, 16)] = vals
                pltpu.make_async_copy(
                    outv.at[b], out_dst(ci), out_sems[b]
                ).start()

                @pl.when(ci + 2 < n_chunks)
                def _():
                    pltpu.make_async_copy(
                        x_src(ci + 2), xv.at[b], in_sems[b]
                    ).start()

            return carry

        lax.fori_loop(0, n_chunks // 2, body, 0)
        for b in range(2):
            pltpu.make_async_copy(
                outv.at[b], out_dst(n_chunks - 2 + b), out_sems[b]
            ).wait()

    return k(wcols, xq)


def kernel(x, weight):
    v, d = weight.shape
    wcols = jnp.zeros((d, 16), jnp.float32).at[:, :v].set(weight.T)
    xq = (
        x.astype(jnp.int32)
        .reshape(_BT, 128, _ST, 8)
        .transpose(2, 0, 3, 1)
    )
    out_q = _embed(xq, wcols)
    return (
        out_q.transpose(2, 4, 1, 3, 0)
        .reshape(16384, 200, d)
    )
